# baseline (speedup 1.0000x reference)
import jax
import jax.numpy as jnp
from jax import lax
from jax.experimental import pallas as pl
from jax.experimental.pallas import tpu as pltpu
from jax.experimental.pallas import tpu_sc as plsc

B, N, D = 1024, 77, 768
NW = 16
BPW = B // NW
CHUNKS = (0, 16, 32, 48, 61)


def _sc_body(tok_ref, emb_ref, pt_ref, pe_ref, out_ref,
             buf, tokv, pev, ptv, insem, outsem, fixsem):
    wid = lax.axis_index("s")
    base = wid * BPW

    pltpu.make_async_copy(tok_ref.at[pl.ds(base, BPW)], tokv, insem.at[0]).start()
    pltpu.make_async_copy(pe_ref, pev, outsem.at[0]).start()
    pltpu.make_async_copy(pt_ref, ptv, fixsem).start()
    pltpu.make_async_copy(tok_ref.at[pl.ds(base, BPW)], tokv, insem.at[0]).wait()
    pltpu.make_async_copy(pe_ref, pev, outsem.at[0]).wait()
    pltpu.make_async_copy(pt_ref, ptv, fixsem).wait()

    NCH = 3 * BPW

    def _src(c):
        b = base + c // 3
        return emb_ref.at[b, :, pl.ds((c % 3) * 256, 256)]

    def _dst(c):
        b = base + c // 3
        return out_ref.at[b, :, pl.ds((c % 3) * 256, 256)]

    def _buf(c):
        return buf.at[c % 4]

    def start_in(c):
        pltpu.make_async_copy(_src(c), _buf(c), insem.at[c % 4]).start()

    def wait_in(c):
        pltpu.make_async_copy(_src(c), _buf(c), insem.at[c % 4]).wait()

    def start_out(c):
        pltpu.make_async_copy(_buf(c), _dst(c), outsem.at[c % 4]).start()

    def wait_out(c):
        pltpu.make_async_copy(_buf(c), _dst(c), outsem.at[c % 4]).wait()

    start_in(0)
    start_in(1)
    for c in range(NCH):
        wait_in(c)
        start_out(c)
        if c + 2 < NCH:
            if c >= 2:
                wait_out(c - 2)
            start_in(c + 2)
    wait_out(NCH - 2)
    wait_out(NCH - 1)

    pt = ptv[...]
    lanes = lax.iota(jnp.int32, 16)

    def fix_row(i, carry):
        for start in CHUNKS:
            m0 = jnp.where(tokv[i, pl.ds(start, 16)] == pt, 1, 0)

            def cond(m):
                return jnp.max(m) > 0

            def body(m):
                inv = jnp.max(jnp.where(m > 0, 16 - lanes, 0))
                n = start + 16 - inv
                cp = pltpu.make_async_copy(
                    pev, out_ref.at[base + i, n], fixsem)
                cp.start()
                cp.wait()
                return jnp.where(lanes == (16 - inv), 0, m)

            lax.while_loop(cond, body, m0)
        return carry

    lax.fori_loop(0, BPW, fix_row, 0)


def sc_kernel(tokenized_text, embedded_text, placeholder_token, placeholder_embedding):
    pt_arr = jnp.full((16,), placeholder_token, jnp.int32)
    mesh = plsc.VectorSubcoreMesh(
        core_axis_name="c", subcore_axis_name="s",
        num_cores=1, num_subcores=16)
    k = pl.kernel(
        _sc_body,
        out_type=jax.ShapeDtypeStruct((B, N, D), jnp.float32),
        mesh=mesh,
        compiler_params=pltpu.CompilerParams(needs_layout_passes=False),
        scratch_types=[
            pltpu.VMEM((4, N, 256), jnp.float32),
            pltpu.VMEM((BPW, N), jnp.int32),
            pltpu.VMEM((D,), jnp.float32),
            pltpu.VMEM((16,), jnp.int32),
            pltpu.SemaphoreType.DMA((4,)),
            pltpu.SemaphoreType.DMA((4,)),
            pltpu.SemaphoreType.DMA,
        ],
    )
    return k(tokenized_text, embedded_text, pt_arr, placeholder_embedding)


def kernel(tokenized_text, embedded_text, placeholder_token, placeholder_embedding):
    return sc_kernel(tokenized_text, embedded_text, placeholder_token,
                     placeholder_embedding)


# speedup vs baseline: 1.1281x; 1.1281x over previous
import jax
import jax.numpy as jnp
from jax import lax
from jax.experimental import pallas as pl
from jax.experimental.pallas import tpu as pltpu
from jax.experimental.pallas import tpu_sc as plsc

B, N, D = 1024, 77, 768
NW = 32
BPW = B // NW
CHUNKS = (0, 16, 32, 48, 61)


def _sc_body(tok_ref, emb_ref, pt_ref, pe_ref, out_ref,
             buf, tokv, pev, ptv, insem, outsem, fixsem):
    wid = lax.axis_index("s") * 2 + lax.axis_index("c")
    base = wid * BPW

    pltpu.make_async_copy(tok_ref.at[pl.ds(base, BPW)], tokv, insem.at[0]).start()
    pltpu.make_async_copy(pe_ref, pev, outsem.at[0]).start()
    pltpu.make_async_copy(pt_ref, ptv, fixsem).start()
    pltpu.make_async_copy(tok_ref.at[pl.ds(base, BPW)], tokv, insem.at[0]).wait()
    pltpu.make_async_copy(pe_ref, pev, outsem.at[0]).wait()
    pltpu.make_async_copy(pt_ref, ptv, fixsem).wait()

    NCH = 2 * BPW

    def _src(c):
        b = base + c // 2
        return emb_ref.at[b, :, pl.ds((c % 2) * 384, 384)]

    def _dst(c):
        b = base + c // 2
        return out_ref.at[b, :, pl.ds((c % 2) * 384, 384)]

    def _buf(c):
        return buf.at[c % 4]

    def start_in(c):
        pltpu.make_async_copy(_src(c), _buf(c), insem.at[c % 4]).start()

    def wait_in(c):
        pltpu.make_async_copy(_src(c), _buf(c), insem.at[c % 4]).wait()

    def start_out(c):
        pltpu.make_async_copy(_buf(c), _dst(c), outsem.at[c % 4]).start()

    def wait_out(c):
        pltpu.make_async_copy(_buf(c), _dst(c), outsem.at[c % 4]).wait()

    start_in(0)
    start_in(1)
    for c in range(NCH):
        wait_in(c)
        start_out(c)
        if c + 2 < NCH:
            if c >= 2:
                wait_out(c - 2)
            start_in(c + 2)
    wait_out(NCH - 2)
    wait_out(NCH - 1)

    pt = ptv[...]
    lanes = lax.iota(jnp.int32, 16)

    def fix_row(i, carry):
        for start in CHUNKS:
            m0 = jnp.where(tokv[i, pl.ds(start, 16)] == pt, 1, 0)

            def cond(m):
                return jnp.max(m) > 0

            def body(m):
                inv = jnp.max(jnp.where(m > 0, 16 - lanes, 0))
                n = start + 16 - inv
                cp = pltpu.make_async_copy(
                    pev, out_ref.at[base + i, n], fixsem)
                cp.start()
                cp.wait()
                return jnp.where(lanes == (16 - inv), 0, m)

            lax.while_loop(cond, body, m0)
        return carry

    lax.fori_loop(0, BPW, fix_row, 0)


def sc_kernel(tokenized_text, embedded_text, placeholder_token, placeholder_embedding):
    pt_arr = jnp.full((16,), placeholder_token, jnp.int32)
    mesh = plsc.VectorSubcoreMesh(
        core_axis_name="c", subcore_axis_name="s",
        num_cores=2, num_subcores=16)
    k = pl.kernel(
        _sc_body,
        out_type=jax.ShapeDtypeStruct((B, N, D), jnp.float32),
        mesh=mesh,
        compiler_params=pltpu.CompilerParams(needs_layout_passes=False),
        scratch_types=[
            pltpu.VMEM((4, N, 384), jnp.float32),
            pltpu.VMEM((BPW, N), jnp.int32),
            pltpu.VMEM((D,), jnp.float32),
            pltpu.VMEM((16,), jnp.int32),
            pltpu.SemaphoreType.DMA((4,)),
            pltpu.SemaphoreType.DMA((4,)),
            pltpu.SemaphoreType.DMA,
        ],
    )
    return k(tokenized_text, embedded_text, pt_arr, placeholder_embedding)


def kernel(tokenized_text, embedded_text, placeholder_token, placeholder_embedding):
    return sc_kernel(tokenized_text, embedded_text, placeholder_token,
                     placeholder_embedding)
